# baseline (speedup 1.0000x reference)
import functools

import jax
import jax.numpy as jnp
from jax import lax
from jax.experimental import pallas as pl
from jax.experimental.pallas import tpu as pltpu
from jax.experimental.pallas import tpu_sc as plsc

N_USERS = 10000
N_NODES = 50000
N_EDGES = 800000
EMB_DIM = 64
BATCH = 1024

EW = 128
NSUB = 16
NCORE = 2
NWIN = 6336
E_PAD = NWIN * EW


N_P = 51200
STRIPE = N_P // NSUB


def _spmm_sc(xa, xb, packed, zeros, d, feature_split, G):
    n = N_P
    mesh = plsc.VectorSubcoreMesh(core_axis_name="c", subcore_axis_name="s")

    if feature_split:
        wps = NWIN // NSUB
    else:
        wps = NWIN // NCORE // NSUB
    npairs = wps // (2 * G)
    assert npairs * 2 * G == wps

    @functools.partial(
        pl.kernel,
        out_type=jax.ShapeDtypeStruct((2 * n, d), jnp.float32),
        mesh=mesh,
        scratch_types=[
            pltpu.VMEM((G * 3, EW), jnp.int32),
            pltpu.VMEM((G * EW, d), jnp.float32),
            pltpu.VMEM((G * 3, EW), jnp.int32),
            pltpu.VMEM((G * EW, d), jnp.float32),
            pltpu.VMEM_SHARED((n, d), jnp.float32),
            pltpu.SemaphoreType.DMA,
            pltpu.SemaphoreType.DMA,
            pltpu.SemaphoreType.DMA,
            pltpu.SemaphoreType.DMA,
        ],
        compiler_params=pltpu.CompilerParams(use_tc_tiling_on_sc=False,
                                             needs_layout_passes=False),
    )
    def spmm(xa_hbm, xb_hbm, pk_hbm, z_hbm, out_hbm,
             pkA, rowsA, pkB, rowsB,
             side, gsemA, ssemA, gsemB, ssemB):
        c = lax.axis_index("c")
        s = lax.axis_index("s")

        pltpu.sync_copy(z_hbm.at[pl.ds(s * STRIPE, STRIPE)],
                        side.at[pl.ds(s * STRIPE, STRIPE)])
        plsc.subcore_barrier()

        if feature_split:
            base_w = s * wps
        else:
            base_w = c * (NWIN // NCORE) + s * wps

        def do_load(S, gi):
            pk, rows, gsem, _ = S
            w0 = base_w + gi * G
            pltpu.sync_copy(pk_hbm.at[pl.ds(3 * w0, 3 * G)], pk)

            @pl.when(c == 0)
            def _():
                for j in range(G):
                    pltpu.async_copy(xa_hbm.at[pk.at[3 * j]],
                                     rows.at[pl.ds(j * EW, EW)], gsem)

            @pl.when(c == 1)
            def _():
                for j in range(G):
                    pltpu.async_copy(xb_hbm.at[pk.at[3 * j]],
                                     rows.at[pl.ds(j * EW, EW)], gsem)

        def drain_gathers(S):
            pk, rows, gsem, _ = S

            @pl.when(c == 0)
            def _():
                for j in range(G):
                    pltpu.make_async_copy(
                        xa_hbm.at[pk.at[3 * j]],
                        rows.at[pl.ds(j * EW, EW)], gsem).wait()

            @pl.when(c == 1)
            def _():
                for j in range(G):
                    pltpu.make_async_copy(
                        xb_hbm.at[pk.at[3 * j]],
                        rows.at[pl.ds(j * EW, EW)], gsem).wait()

        def do_scale(S):
            pk, rows = S[0], S[1]

            @plsc.parallel_loop(0, G * (EW // 16), unroll=4)
            def _(q):
                j = q // (EW // 16)
                cc = q % (EW // 16)
                ev16 = plsc.bitcast(pk[3 * j + 2, pl.ds(cc * 16, 16)],
                                    jnp.float32)
                r0 = j * EW + cc * 16
                for l in range(16):
                    sv = ev16[l]
                    for jj in range(d // 16):
                        sl = pl.ds(jj * 16, 16)
                        rows[r0 + l, sl] = rows[r0 + l, sl] * sv

        def do_fire(S):
            pk, rows, _, ssem = S
            for j in range(G):
                pltpu.async_copy(rows.at[pl.ds(j * EW, EW)],
                                 side.at[pk.at[3 * j + 1]], ssem, add=True)

        def drain_scatters(S):
            pk, rows, _, ssem = S
            for j in range(G):
                pltpu.make_async_copy(rows.at[pl.ds(j * EW, EW)],
                                      side.at[pk.at[3 * j + 1]], ssem).wait()

        A = (pkA, rowsA, gsemA, ssemA)
        B = (pkB, rowsB, gsemB, ssemB)
        do_load(A, 0)

        @pl.loop(0, npairs)
        def _(p):
            drain_gathers(A)
            do_scale(A)

            @pl.when(p > 0)
            def _():
                drain_scatters(B)

            do_fire(A)
            do_load(B, 2 * p + 1)
            drain_gathers(B)
            do_scale(B)
            drain_scatters(A)
            do_fire(B)

            @pl.when(p < npairs - 1)
            def _():
                do_load(A, 2 * p + 2)

        drain_scatters(B)
        plsc.subcore_barrier()
        pltpu.sync_copy(side.at[pl.ds(s * STRIPE, STRIPE)],
                        out_hbm.at[pl.ds(c * n + s * STRIPE, STRIPE)])

    return spmm(xa, xb, packed, zeros)


def _prep_x0(xcat_p):
    R = 6400
    nblk = N_P // R

    def body(x_ref, lo_ref, hi_ref):
        v = x_ref[...]
        lo_ref[...] = v[:, :32]
        hi_ref[...] = v[:, 32:]

    f32 = jnp.float32
    return pl.pallas_call(
        body,
        grid=(nblk,),
        in_specs=[pl.BlockSpec((R, EMB_DIM), lambda i: (i, 0))],
        out_specs=[
            pl.BlockSpec((R, 32), lambda i: (i, 0)),
            pl.BlockSpec((R, 32), lambda i: (i, 0)),
        ],
        out_shape=[
            jax.ShapeDtypeStruct((N_P, 32), f32),
            jax.ShapeDtypeStruct((N_P, 32), f32),
        ],
    )(xcat_p)


def _dense_tc(x, side2, Wg, bg, Wb, bb, concat_mode, pad_to,
              emit_halves=False):
    n, d_in = x.shape
    d_out = Wg.shape[1]
    R = 6400
    nblk = n // R
    d_side = side2.shape[1]

    def body(x_ref, s0_ref, s1_ref, wg_ref, bg_ref, wb_ref, bb_ref, *outs):
        xb = x_ref[...]
        if concat_mode:
            side = jnp.concatenate([s0_ref[...], s1_ref[...]], axis=1)
        else:
            side = s0_ref[...] + s1_ref[...]
        a = jnp.dot(xb + side, wg_ref[...],
                    preferred_element_type=jnp.float32) + bg_ref[...]
        a = jnp.where(a >= 0, a, 0.01 * a)
        b = jnp.dot(xb * side, wb_ref[...],
                    preferred_element_type=jnp.float32) + bb_ref[...]
        b = jnp.where(b >= 0, b, 0.01 * b)
        y = a + b
        nrm = jnp.sqrt(jnp.sum(y * y, axis=1, keepdims=True))
        y = y / jnp.maximum(nrm, 1e-12)
        if emit_halves:
            outs[1][...] = y[:, :d_out // 2]
            outs[2][...] = y[:, d_out // 2:]
        if pad_to > d_out:
            y = jnp.concatenate(
                [y, jnp.zeros((y.shape[0], pad_to - d_out), jnp.float32)],
                axis=1)
        outs[0][...] = y

    f32 = jnp.float32
    out_specs = [pl.BlockSpec((R, pad_to), lambda i: (i, 0))]
    out_shape = [jax.ShapeDtypeStruct((n, pad_to), f32)]
    if emit_halves:
        for _ in range(2):
            out_specs.append(pl.BlockSpec((R, d_out // 2), lambda i: (i, 0)))
            out_shape.append(jax.ShapeDtypeStruct((n, d_out // 2), f32))
    return pl.pallas_call(
        body,
        grid=(nblk,),
        in_specs=[
            pl.BlockSpec((R, d_in), lambda i: (i, 0)),
            pl.BlockSpec((R, d_side), lambda i: (i, 0)),
            pl.BlockSpec((R, d_side), lambda i: (i + nblk, 0)),
            pl.BlockSpec((d_in, d_out), lambda i: (0, 0)),
            pl.BlockSpec((1, d_out), lambda i: (0, 0)),
            pl.BlockSpec((d_in, d_out), lambda i: (0, 0)),
            pl.BlockSpec((1, d_out), lambda i: (0, 0)),
        ],
        out_specs=out_specs,
        out_shape=out_shape,
    )(x, side2, side2, Wg, bg, Wb, bb)


def _gather_sc(x0, x1, x2, x3, users, items):
    per_w = BATCH // (NCORE * NSUB)
    mesh = plsc.VectorSubcoreMesh(core_axis_name="c", subcore_axis_name="s")
    f32 = jnp.float32
    out_types = tuple(
        jax.ShapeDtypeStruct((BATCH, dd), f32) for dd in (64, 32, 16, 16)
    ) * 2

    @functools.partial(
        pl.kernel,
        out_type=out_types,
        mesh=mesh,
        scratch_types=[
            pltpu.VMEM((1, per_w), jnp.int32),
            pltpu.VMEM((per_w, 64), f32),
            pltpu.VMEM((per_w, 32), f32),
            pltpu.VMEM((per_w, 16), f32),
            pltpu.VMEM((per_w, 16), f32),
        ],
        compiler_params=pltpu.CompilerParams(use_tc_tiling_on_sc=False),
    )
    def gat(x0_hbm, x1_hbm, x2_hbm, x3_hbm, u_hbm, i_hbm,
            u0, u1, u2, u3, i0, i1, i2, i3,
            idx, r0, r1, r2, r3):
        c = lax.axis_index("c")
        s = lax.axis_index("s")
        wid = s * NCORE + c
        base = wid * per_w

        def do(ind_hbm, off, o0, o1, o2, o3):
            pltpu.sync_copy(ind_hbm.at[pl.ds(base, per_w)], idx.at[0])
            if off:
                for j in range(per_w // 16):
                    sl = pl.ds(j * 16, 16)
                    idx[0, sl] = idx[0, sl] + off
            pltpu.sync_copy(x0_hbm.at[idx.at[0]], r0)
            pltpu.sync_copy(r0, o0.at[pl.ds(base, per_w)])
            pltpu.sync_copy(x1_hbm.at[idx.at[0]], r1)
            pltpu.sync_copy(r1, o1.at[pl.ds(base, per_w)])
            pltpu.sync_copy(x2_hbm.at[idx.at[0]], r2)
            pltpu.sync_copy(r2, o2.at[pl.ds(base, per_w)])
            pltpu.sync_copy(x3_hbm.at[idx.at[0]], r3)
            pltpu.sync_copy(r3, o3.at[pl.ds(base, per_w)])

        do(u_hbm, 0, u0, u1, u2, u3)
        do(i_hbm, N_USERS, i0, i1, i2, i3)

    return gat(x0, x1, x2, x3, users, items)


def _dot_tc(parts):
    def body(*refs):
        o_ref = refs[-1]
        acc = None
        for k in range(4):
            t = jnp.sum(refs[k][...] * refs[k + 4][...], axis=1,
                        keepdims=True)
            acc = t if acc is None else acc + t
        o_ref[...] = acc

    out = pl.pallas_call(
        body,
        out_shape=jax.ShapeDtypeStruct((BATCH, 1), jnp.float32),
    )(*parts)
    return out.reshape(BATCH)


def kernel(edge_vals, user_embed, entity_embed,
           W_gc_0, b_gc_0, W_bi_0, b_bi_0,
           W_gc_1, b_gc_1, W_bi_1, b_bi_1,
           W_gc_2, b_gc_2, W_bi_2, b_bi_2,
           edge_index, users, pos_items):
    f32 = jnp.float32
    npad = E_PAD - N_EDGES
    spread = (jnp.arange(npad, dtype=jnp.int32) * 16) % N_NODES
    dst2d = jnp.concatenate([edge_index[0], spread]).reshape(NWIN, EW)
    src2d = jnp.concatenate([edge_index[1], spread]).reshape(NWIN, EW)
    ev2d = jax.lax.bitcast_convert_type(
        jnp.concatenate([edge_vals, jnp.zeros((npad,), f32)]),
        jnp.int32).reshape(NWIN, EW)
    packed = jnp.stack([src2d, dst2d, ev2d], axis=1).reshape(NWIN * 3, EW)
    z32 = jnp.zeros((N_P, 32), f32)
    z16 = jnp.zeros((N_P, 16), f32)
    n = N_P

    xcat_p = jnp.concatenate(
        [user_embed, entity_embed,
         jnp.zeros((N_P - N_NODES, EMB_DIM), f32)], axis=0)
    lo_q, hi_q = _prep_x0(xcat_p)

    side0 = _spmm_sc(lo_q, hi_q, packed, z32, 32, feature_split=True, G=3)
    x1, x1_lo, x1_hi = _dense_tc(
        xcat_p, side0, W_gc_0, b_gc_0, W_bi_0, b_bi_0,
        concat_mode=True, pad_to=32, emit_halves=True)

    side1 = _spmm_sc(x1_lo, x1_hi, packed, z16, 16, feature_split=True, G=11)
    (x2,) = _dense_tc(
        x1, side1, W_gc_1, b_gc_1, W_bi_1, b_bi_1,
        concat_mode=True, pad_to=16)

    side2 = _spmm_sc(x2, x2, packed, z16, 16, feature_split=False, G=11)
    (x3,) = _dense_tc(
        x2, side2, W_gc_2, b_gc_2, W_bi_2, b_bi_2,
        concat_mode=False, pad_to=16)

    parts = _gather_sc(xcat_p, x1, x2, x3, users, pos_items)
    return _dot_tc(parts)


# speedup vs baseline: 1.0840x; 1.0840x over previous
import functools

import jax
import jax.numpy as jnp
from jax import lax
from jax.experimental import pallas as pl
from jax.experimental.pallas import tpu as pltpu
from jax.experimental.pallas import tpu_sc as plsc

N_USERS = 10000
N_NODES = 50000
N_EDGES = 800000
EMB_DIM = 64
BATCH = 1024

EW = 128
NSUB = 16
NCORE = 2
NWIN = 6336
E_PAD = NWIN * EW


N_P = 51200
STRIPE = N_P // NSUB


def _spmm_sc(xa, xb, packed, zeros, d, feature_split, G):
    n = N_P
    mesh = plsc.VectorSubcoreMesh(core_axis_name="c", subcore_axis_name="s")

    if feature_split:
        wps = NWIN // NSUB
    else:
        wps = NWIN // NCORE // NSUB
    npairs = wps // (2 * G)
    assert npairs * 2 * G == wps

    @functools.partial(
        pl.kernel,
        out_type=jax.ShapeDtypeStruct((2 * n, d), jnp.float32),
        mesh=mesh,
        scratch_types=[
            pltpu.VMEM((G * 3, EW), jnp.int32),
            pltpu.VMEM((G * EW, d), jnp.float32),
            pltpu.VMEM((G * 3, EW), jnp.int32),
            pltpu.VMEM((G * EW, d), jnp.float32),
            pltpu.VMEM_SHARED((n, d), jnp.float32),
            pltpu.SemaphoreType.DMA,
            pltpu.SemaphoreType.DMA,
            pltpu.SemaphoreType.DMA,
            pltpu.SemaphoreType.DMA,
        ],
        compiler_params=pltpu.CompilerParams(use_tc_tiling_on_sc=False,
                                             needs_layout_passes=False),
    )
    def spmm(xa_hbm, xb_hbm, pk_hbm, z_hbm, out_hbm,
             pkA, rowsA, pkB, rowsB,
             side, gsemA, ssemA, gsemB, ssemB):
        c = lax.axis_index("c")
        s = lax.axis_index("s")

        pltpu.sync_copy(z_hbm.at[pl.ds(s * STRIPE, STRIPE)],
                        side.at[pl.ds(s * STRIPE, STRIPE)])
        plsc.subcore_barrier()

        if feature_split:
            base_w = s * wps
        else:
            base_w = c * (NWIN // NCORE) + s * wps

        def do_load(S, gi):
            pk, rows, gsem, _ = S
            w0 = base_w + gi * G
            pltpu.sync_copy(pk_hbm.at[pl.ds(3 * w0, 3 * G)], pk)

            @pl.when(c == 0)
            def _():
                for j in range(G):
                    pltpu.async_copy(xa_hbm.at[pk.at[3 * j]],
                                     rows.at[pl.ds(j * EW, EW)], gsem)

            @pl.when(c == 1)
            def _():
                for j in range(G):
                    pltpu.async_copy(xb_hbm.at[pk.at[3 * j]],
                                     rows.at[pl.ds(j * EW, EW)], gsem)

        def drain_gathers(S):
            pk, rows, gsem, _ = S

            @pl.when(c == 0)
            def _():
                for j in range(G):
                    pltpu.make_async_copy(
                        xa_hbm.at[pk.at[3 * j]],
                        rows.at[pl.ds(j * EW, EW)], gsem).wait()

            @pl.when(c == 1)
            def _():
                for j in range(G):
                    pltpu.make_async_copy(
                        xb_hbm.at[pk.at[3 * j]],
                        rows.at[pl.ds(j * EW, EW)], gsem).wait()

        def do_scale(S):
            pk, rows = S[0], S[1]

            @plsc.parallel_loop(0, G * (EW // 16), unroll=2)
            def _(q):
                j = q // (EW // 16)
                cc = q % (EW // 16)
                ev16 = plsc.bitcast(pk[3 * j + 2, pl.ds(cc * 16, 16)],
                                    jnp.float32)
                r0 = j * EW + cc * 16
                for l in range(16):
                    sv = ev16[l]
                    for jj in range(d // 16):
                        sl = pl.ds(jj * 16, 16)
                        rows[r0 + l, sl] = rows[r0 + l, sl] * sv

        def do_fire(S):
            pk, rows, _, ssem = S
            for j in range(G):
                pltpu.async_copy(rows.at[pl.ds(j * EW, EW)],
                                 side.at[pk.at[3 * j + 1]], ssem, add=True)

        def drain_scatters(S):
            pk, rows, _, ssem = S
            for j in range(G):
                pltpu.make_async_copy(rows.at[pl.ds(j * EW, EW)],
                                      side.at[pk.at[3 * j + 1]], ssem).wait()

        A = (pkA, rowsA, gsemA, ssemA)
        B = (pkB, rowsB, gsemB, ssemB)
        do_load(A, 0)

        @pl.loop(0, npairs)
        def _(p):
            drain_gathers(A)
            do_scale(A)

            @pl.when(p > 0)
            def _():
                drain_scatters(B)

            do_fire(A)
            do_load(B, 2 * p + 1)
            drain_gathers(B)
            do_scale(B)
            drain_scatters(A)
            do_fire(B)

            @pl.when(p < npairs - 1)
            def _():
                do_load(A, 2 * p + 2)

        drain_scatters(B)
        plsc.subcore_barrier()
        pltpu.sync_copy(side.at[pl.ds(s * STRIPE, STRIPE)],
                        out_hbm.at[pl.ds(c * n + s * STRIPE, STRIPE)])

    return spmm(xa, xb, packed, zeros)


def _prep_x0(xcat_p):
    R = 6400
    nblk = N_P // R

    def body(x_ref, lo_ref, hi_ref):
        v = x_ref[...]
        lo_ref[...] = v[:, :32]
        hi_ref[...] = v[:, 32:]

    f32 = jnp.float32
    return pl.pallas_call(
        body,
        grid=(nblk,),
        in_specs=[pl.BlockSpec((R, EMB_DIM), lambda i: (i, 0))],
        out_specs=[
            pl.BlockSpec((R, 32), lambda i: (i, 0)),
            pl.BlockSpec((R, 32), lambda i: (i, 0)),
        ],
        out_shape=[
            jax.ShapeDtypeStruct((N_P, 32), f32),
            jax.ShapeDtypeStruct((N_P, 32), f32),
        ],
    )(xcat_p)


def _dense_tc(x, side2, Wg, bg, Wb, bb, concat_mode, pad_to,
              emit_halves=False):
    n, d_in = x.shape
    d_out = Wg.shape[1]
    R = 6400
    nblk = n // R
    d_side = side2.shape[1]

    def body(x_ref, s0_ref, s1_ref, wg_ref, bg_ref, wb_ref, bb_ref, *outs):
        xb = x_ref[...]
        if concat_mode:
            side = jnp.concatenate([s0_ref[...], s1_ref[...]], axis=1)
        else:
            side = s0_ref[...] + s1_ref[...]
        a = jnp.dot(xb + side, wg_ref[...],
                    preferred_element_type=jnp.float32) + bg_ref[...]
        a = jnp.where(a >= 0, a, 0.01 * a)
        b = jnp.dot(xb * side, wb_ref[...],
                    preferred_element_type=jnp.float32) + bb_ref[...]
        b = jnp.where(b >= 0, b, 0.01 * b)
        y = a + b
        nrm = jnp.sqrt(jnp.sum(y * y, axis=1, keepdims=True))
        y = y / jnp.maximum(nrm, 1e-12)
        if emit_halves:
            outs[1][...] = y[:, :d_out // 2]
            outs[2][...] = y[:, d_out // 2:]
        if pad_to > d_out:
            y = jnp.concatenate(
                [y, jnp.zeros((y.shape[0], pad_to - d_out), jnp.float32)],
                axis=1)
        outs[0][...] = y

    f32 = jnp.float32
    out_specs = [pl.BlockSpec((R, pad_to), lambda i: (i, 0))]
    out_shape = [jax.ShapeDtypeStruct((n, pad_to), f32)]
    if emit_halves:
        for _ in range(2):
            out_specs.append(pl.BlockSpec((R, d_out // 2), lambda i: (i, 0)))
            out_shape.append(jax.ShapeDtypeStruct((n, d_out // 2), f32))
    return pl.pallas_call(
        body,
        grid=(nblk,),
        in_specs=[
            pl.BlockSpec((R, d_in), lambda i: (i, 0)),
            pl.BlockSpec((R, d_side), lambda i: (i, 0)),
            pl.BlockSpec((R, d_side), lambda i: (i + nblk, 0)),
            pl.BlockSpec((d_in, d_out), lambda i: (0, 0)),
            pl.BlockSpec((1, d_out), lambda i: (0, 0)),
            pl.BlockSpec((d_in, d_out), lambda i: (0, 0)),
            pl.BlockSpec((1, d_out), lambda i: (0, 0)),
        ],
        out_specs=out_specs,
        out_shape=out_shape,
    )(x, side2, side2, Wg, bg, Wb, bb)


def _gather_sc(x0, x1, x2, x3, users, items):
    per_w = BATCH // (NCORE * NSUB)
    mesh = plsc.VectorSubcoreMesh(core_axis_name="c", subcore_axis_name="s")
    f32 = jnp.float32
    out_types = tuple(
        jax.ShapeDtypeStruct((BATCH, dd), f32) for dd in (64, 32, 16, 16)
    ) * 2

    @functools.partial(
        pl.kernel,
        out_type=out_types,
        mesh=mesh,
        scratch_types=[
            pltpu.VMEM((1, per_w), jnp.int32),
            pltpu.VMEM((per_w, 64), f32),
            pltpu.VMEM((per_w, 32), f32),
            pltpu.VMEM((per_w, 16), f32),
            pltpu.VMEM((per_w, 16), f32),
        ],
        compiler_params=pltpu.CompilerParams(use_tc_tiling_on_sc=False),
    )
    def gat(x0_hbm, x1_hbm, x2_hbm, x3_hbm, u_hbm, i_hbm,
            u0, u1, u2, u3, i0, i1, i2, i3,
            idx, r0, r1, r2, r3):
        c = lax.axis_index("c")
        s = lax.axis_index("s")
        wid = s * NCORE + c
        base = wid * per_w

        def do(ind_hbm, off, o0, o1, o2, o3):
            pltpu.sync_copy(ind_hbm.at[pl.ds(base, per_w)], idx.at[0])
            if off:
                for j in range(per_w // 16):
                    sl = pl.ds(j * 16, 16)
                    idx[0, sl] = idx[0, sl] + off
            pltpu.sync_copy(x0_hbm.at[idx.at[0]], r0)
            pltpu.sync_copy(r0, o0.at[pl.ds(base, per_w)])
            pltpu.sync_copy(x1_hbm.at[idx.at[0]], r1)
            pltpu.sync_copy(r1, o1.at[pl.ds(base, per_w)])
            pltpu.sync_copy(x2_hbm.at[idx.at[0]], r2)
            pltpu.sync_copy(r2, o2.at[pl.ds(base, per_w)])
            pltpu.sync_copy(x3_hbm.at[idx.at[0]], r3)
            pltpu.sync_copy(r3, o3.at[pl.ds(base, per_w)])

        do(u_hbm, 0, u0, u1, u2, u3)
        do(i_hbm, N_USERS, i0, i1, i2, i3)

    return gat(x0, x1, x2, x3, users, items)


def _dot_tc(parts):
    def body(*refs):
        o_ref = refs[-1]
        acc = None
        for k in range(4):
            t = jnp.sum(refs[k][...] * refs[k + 4][...], axis=1,
                        keepdims=True)
            acc = t if acc is None else acc + t
        o_ref[...] = acc

    out = pl.pallas_call(
        body,
        out_shape=jax.ShapeDtypeStruct((BATCH, 1), jnp.float32),
    )(*parts)
    return out.reshape(BATCH)


def kernel(edge_vals, user_embed, entity_embed,
           W_gc_0, b_gc_0, W_bi_0, b_bi_0,
           W_gc_1, b_gc_1, W_bi_1, b_bi_1,
           W_gc_2, b_gc_2, W_bi_2, b_bi_2,
           edge_index, users, pos_items):
    f32 = jnp.float32
    npad = E_PAD - N_EDGES
    spread = (jnp.arange(npad, dtype=jnp.int32) * 16) % N_NODES
    dst2d = jnp.concatenate([edge_index[0], spread]).reshape(NWIN, EW)
    src2d = jnp.concatenate([edge_index[1], spread]).reshape(NWIN, EW)
    ev2d = jax.lax.bitcast_convert_type(
        jnp.concatenate([edge_vals, jnp.zeros((npad,), f32)]),
        jnp.int32).reshape(NWIN, EW)
    packed = jnp.stack([src2d, dst2d, ev2d], axis=1).reshape(NWIN * 3, EW)
    z32 = jnp.zeros((N_P, 32), f32)
    z16 = jnp.zeros((N_P, 16), f32)
    n = N_P

    xcat_p = jnp.concatenate(
        [user_embed, entity_embed,
         jnp.zeros((N_P - N_NODES, EMB_DIM), f32)], axis=0)
    lo_q, hi_q = _prep_x0(xcat_p)

    side0 = _spmm_sc(lo_q, hi_q, packed, z32, 32, feature_split=True, G=3)
    x1, x1_lo, x1_hi = _dense_tc(
        xcat_p, side0, W_gc_0, b_gc_0, W_bi_0, b_bi_0,
        concat_mode=True, pad_to=32, emit_halves=True)

    side1 = _spmm_sc(x1_lo, x1_hi, packed, z16, 16, feature_split=True, G=11)
    (x2,) = _dense_tc(
        x1, side1, W_gc_1, b_gc_1, W_bi_1, b_bi_1,
        concat_mode=True, pad_to=16)

    side2 = _spmm_sc(x2, x2, packed, z16, 16, feature_split=False, G=11)
    (x3,) = _dense_tc(
        x2, side2, W_gc_2, b_gc_2, W_bi_2, b_bi_2,
        concat_mode=False, pad_to=16)

    parts = _gather_sc(xcat_p, x1, x2, x3, users, pos_items)
    return _dot_tc(parts)
